# baseline (speedup 1.0000x reference)
import functools

import jax
import jax.numpy as jnp
from jax import lax
from jax.experimental import pallas as pl
from jax.experimental.pallas import tpu as pltpu
from jax.experimental.pallas import tpu_sc as plsc

MEM_ROWS = 100000
CHUNK = 1000
NBUF = 10
NGROUPS = MEM_ROWS // (CHUNK * NBUF)
KDIM = 128
VDIM = 512

NW = 32
SC_ROWS_PER_W = 3072
SC_CHUNK = 96
SC_NIT = SC_ROWS_PER_W // SC_CHUNK

_mesh = plsc.VectorSubcoreMesh(core_axis_name="c", subcore_axis_name="s")


@functools.partial(
    pl.kernel,
    out_type=jax.ShapeDtypeStruct((NW, VDIM), jnp.float32),
    mesh=_mesh,
    scratch_types=[
        pltpu.VMEM((2, SC_CHUNK, VDIM), jnp.float32),
        pltpu.SemaphoreType.DMA((2,)),
    ],
)
def _sc_stream(v_hbm, out_hbm, vbuf, sems):
    cid = lax.axis_index("c")
    sid = lax.axis_index("s")
    wid = sid * 2 + cid
    base = wid * SC_ROWS_PER_W

    def vcopy(i, b):
        off = pl.multiple_of(base + i * SC_CHUNK, 8)
        return pltpu.make_async_copy(
            v_hbm.at[pl.ds(off, SC_CHUNK), :], vbuf.at[b], sems.at[b]
        )

    vcopy(0, 0).start()
    for i in range(SC_NIT):
        if i + 1 < SC_NIT:
            vcopy(i + 1, (i + 1) % 2).start()
        vcopy(i, i % 2).wait()
    pltpu.sync_copy(vbuf.at[0, 0], out_hbm.at[wid])


def _soft_read_kernel(x_ref, fz_ref, k_hbm, v_hbm, o_ref, *bufs):
    kbufs = bufs[:NBUF]
    vbufs = bufs[NBUF : 2 * NBUF]
    sems = bufs[2 * NBUF]

    def kcopy(i, b):
        return pltpu.make_async_copy(
            k_hbm.at[pl.ds(i * CHUNK, CHUNK), :], kbufs[b], sems.at[b, 0]
        )

    def vcopy(i, b):
        return pltpu.make_async_copy(
            v_hbm.at[pl.ds(i * CHUNK, CHUNK), :], vbufs[b], sems.at[b, 1]
        )

    for b in range(NBUF):
        kcopy(b, b).start()
        vcopy(b, b).start()

    x = x_ref[...]
    x_norm = jnp.sqrt(jnp.sum(x * x))
    ones = jnp.ones((1, KDIM), jnp.float32)

    def group(g, carry):
        acc, den = carry
        for b in range(NBUF):
            i = g * NBUF + b
            kcopy(i, b).wait()
            vcopy(i, b).wait()
            k = kbufs[b][...]
            v = vbufs[b][...]
            kt = k.T
            num = jnp.dot(x, kt)
            sq = jnp.dot(ones, kt * kt)
            denom = jnp.maximum(x_norm * jnp.sqrt(sq), 1e-6)
            p = jnp.exp(num / denom)
            part = jnp.dot(p, v)
            psum = jnp.sum(p)

            @pl.when(i + NBUF < NGROUPS * NBUF)
            def _next():
                kcopy(i + NBUF, b).start()
                vcopy(i + NBUF, b).start()

            acc = acc + part
            den = den + psum
        return (acc, den)

    acc, den = jax.lax.fori_loop(
        0, NGROUPS, group, (jnp.zeros((1, VDIM), jnp.float32), jnp.float32(0.0))
    )
    o_ref[...] = 0.7 * (acc / den) + 0.3 * fz_ref[...]


@jax.jit
def _soft_read(x_key, f_z_value, key_memory, value_memory):
    m, kdim = key_memory.shape
    v2d = value_memory.reshape(m, VDIM)
    fz2d = f_z_value.reshape(1, VDIM)

    sc_parts = _sc_stream(v2d)

    out = pl.pallas_call(
        _soft_read_kernel,
        in_specs=[
            pl.BlockSpec(memory_space=pltpu.MemorySpace.VMEM),
            pl.BlockSpec(memory_space=pltpu.MemorySpace.VMEM),
            pl.BlockSpec(memory_space=pltpu.MemorySpace.HBM),
            pl.BlockSpec(memory_space=pltpu.MemorySpace.HBM),
        ],
        out_specs=pl.BlockSpec(memory_space=pltpu.MemorySpace.VMEM),
        out_shape=jax.ShapeDtypeStruct((1, VDIM), jnp.float32),
        scratch_shapes=(
            [pltpu.VMEM((CHUNK, KDIM), jnp.float32) for _ in range(NBUF)]
            + [pltpu.VMEM((CHUNK, VDIM), jnp.float32) for _ in range(NBUF)]
            + [pltpu.SemaphoreType.DMA((NBUF, 2))]
        ),
    )(x_key, fz2d, key_memory, v2d)
    out = out + 0.0 * jnp.sum(sc_parts)
    return out.reshape(f_z_value.shape)


def kernel(x_key, f_z_value, key_memory, value_memory):
    return _soft_read(x_key, f_z_value, key_memory, value_memory)


# speedup vs baseline: 1.2404x; 1.2404x over previous
import functools

import jax
import jax.numpy as jnp
from jax import lax
from jax.experimental import pallas as pl
from jax.experimental.pallas import tpu as pltpu
from jax.experimental.pallas import tpu_sc as plsc

MEM_ROWS = 100000
KDIM = 128
VDIM = 512

TC_ROWS = 61600
TC_CHUNK = 880
TC_NBUF = 10
TC_NGROUPS = TC_ROWS // (TC_CHUNK * TC_NBUF)

NW = 32
SC_ROWS_PER_W = 1200
SC_CHUNK = 48
SC_NIT = SC_ROWS_PER_W // SC_CHUNK
SC_GROUPS = SC_CHUNK // 16
VSLICES = VDIM // 16

_mesh = plsc.VectorSubcoreMesh(core_axis_name="c", subcore_axis_name="s")


def _newton_rsqrt(x):
    xi = lax.bitcast_convert_type(x, jnp.int32)
    yi = jnp.int32(0x5F3759DF) - (xi >> 1)
    y = lax.bitcast_convert_type(yi, jnp.float32)
    for _ in range(3):
        y = y * (1.5 - 0.5 * x * y * y)
    return y


@functools.partial(
    pl.kernel,
    out_type=(
        jax.ShapeDtypeStruct((NW, VDIM), jnp.float32),
        jax.ShapeDtypeStruct((NW, 128), jnp.float32),
    ),
    mesh=_mesh,
    scratch_types=[
        pltpu.VMEM((SC_CHUNK, KDIM), jnp.float32),
        pltpu.VMEM((SC_CHUNK, KDIM), jnp.float32),
        pltpu.VMEM((SC_CHUNK, VDIM), jnp.float32),
        pltpu.VMEM((SC_CHUNK, VDIM), jnp.float32),
        pltpu.VMEM((1, KDIM), jnp.float32),
        pltpu.VMEM((1, VDIM), jnp.float32),
        pltpu.VMEM((1, 128), jnp.float32),
        pltpu.SemaphoreType.DMA((2, 2)),
    ],
)
def _sc_soft_read(xs_hbm, k_hbm, v_hbm, part_hbm, den_hbm, kbuf0, kbuf1,
                  vbuf0, vbuf1, xsbuf, obuf, dbuf, sems):
    cid = lax.axis_index("c")
    sid = lax.axis_index("s")
    wid = sid * 2 + cid
    base = TC_ROWS + wid * SC_ROWS_PER_W

    pltpu.sync_copy(xs_hbm, xsbuf)

    kbufs = (kbuf0, kbuf1)
    vbufs = (vbuf0, vbuf1)

    def kcopy(i, b):
        off = pl.multiple_of(base + i * SC_CHUNK, 8)
        return pltpu.make_async_copy(
            k_hbm.at[pl.ds(off, SC_CHUNK), :], kbufs[b], sems.at[b, 0]
        )

    def vcopy(i, b):
        off = pl.multiple_of(base + i * SC_CHUNK, 8)
        return pltpu.make_async_copy(
            v_hbm.at[pl.ds(off, SC_CHUNK), :], vbufs[b], sems.at[b, 1]
        )

    kcopy(0, 0).start()
    vcopy(0, 0).start()
    kcopy(1, 1).start()
    vcopy(1, 1).start()

    _gdn = lax.GatherDimensionNumbers(
        offset_dims=(), collapsed_slice_dims=(0,), start_index_map=(0,)
    )

    def _shuffle(v, idx):
        return lax.gather(
            v,
            idx[:, None],
            _gdn,
            slice_sizes=(1,),
            mode=lax.GatherScatterMode.PROMISE_IN_BOUNDS,
        )

    def lane_sum_splat(v):
        idx0 = lax.iota(jnp.int32, 16)
        for k in (1, 2, 4, 8):
            v = v + _shuffle(v, idx0 ^ k)
        return v

    xs = [xsbuf[0, pl.ds(16 * s, 16)] for s in range(KDIM // 16)]
    xn2v = xs[0] * xs[0]
    for s in range(1, KDIM // 16):
        xn2v = xn2v + xs[s] * xs[s]
    inv_xn = _newton_rsqrt(lane_sum_splat(xn2v))

    zero16 = jnp.zeros((16,), jnp.float32)

    def chunk_compute(i, b, carry):
        kb = kbufs[b]
        vb = vbufs[b]
        kcopy(i, b).wait()
        vcopy(i, b).wait()

        def rowloop(j, rc):
            accs, den = rc
            ks = [kb[j, pl.ds(16 * s, 16)] for s in range(KDIM // 16)]
            acc_n = ks[0] * xs[0]
            acc_q = ks[0] * ks[0]
            for s in range(1, KDIM // 16):
                acc_n = acc_n + ks[s] * xs[s]
                acc_q = acc_q + ks[s] * ks[s]
            nv = lane_sum_splat(acc_n)
            qv = lane_sum_splat(acc_q)
            pv = jnp.exp(nv * _newton_rsqrt(qv) * inv_xn)
            den = den + pv
            accs = tuple(
                accs[s] + pv * vb[j, pl.ds(s * 16, 16)] for s in range(VSLICES)
            )
            return (accs, den)

        return lax.fori_loop(0, SC_CHUNK, rowloop, carry)

    def pair_body(gp, carry):
        i0 = gp * 2
        for b in range(2):
            i = i0 + b
            carry = chunk_compute(i, b, carry)

            @pl.when(i + 2 < SC_NIT)
            def _next():
                kcopy(i + 2, b).start()
                vcopy(i + 2, b).start()

        return carry

    carry = lax.fori_loop(
        0, SC_NIT // 2, pair_body, (tuple(zero16 for _ in range(VSLICES)), zero16)
    )
    accs, den = chunk_compute(SC_NIT - 1, 0, carry)

    for s in range(VSLICES):
        obuf[0, pl.ds(s * 16, 16)] = accs[s]
    for s in range(8):
        dbuf[0, pl.ds(s * 16, 16)] = zero16
    dbuf[0, pl.ds(0, 16)] = den * (1.0 / 16.0)
    pltpu.sync_copy(obuf.at[0], part_hbm.at[wid])
    pltpu.sync_copy(dbuf.at[0], den_hbm.at[wid])


def _tc_soft_read_kernel(x_ref, k_hbm, v_hbm, oacc_ref, oden_ref, *bufs):
    kbufs = bufs[:TC_NBUF]
    vbufs = bufs[TC_NBUF : 2 * TC_NBUF]
    sems = bufs[2 * TC_NBUF]

    def kcopy(i, b):
        return pltpu.make_async_copy(
            k_hbm.at[pl.ds(i * TC_CHUNK, TC_CHUNK), :], kbufs[b], sems.at[b, 0]
        )

    def vcopy(i, b):
        return pltpu.make_async_copy(
            v_hbm.at[pl.ds(i * TC_CHUNK, TC_CHUNK), :], vbufs[b], sems.at[b, 1]
        )

    for b in range(TC_NBUF):
        kcopy(b, b).start()
        vcopy(b, b).start()

    x = x_ref[...]
    x_norm = jnp.sqrt(jnp.sum(x * x))
    ones = jnp.ones((1, KDIM), jnp.float32)

    def group(g, carry):
        acc, den = carry
        for b in range(TC_NBUF):
            i = g * TC_NBUF + b
            kcopy(i, b).wait()
            vcopy(i, b).wait()
            k = kbufs[b][...]
            v = vbufs[b][...]
            kt = k.T
            num = jnp.dot(x, kt)
            sq = jnp.dot(ones, kt * kt)
            denom = jnp.maximum(x_norm * jnp.sqrt(sq), 1e-6)
            p = jnp.exp(num / denom)
            part = jnp.dot(p, v)
            psum = jnp.sum(p)

            @pl.when(i + TC_NBUF < TC_NGROUPS * TC_NBUF)
            def _next():
                kcopy(i + TC_NBUF, b).start()
                vcopy(i + TC_NBUF, b).start()

            acc = acc + part
            den = den + psum
        return (acc, den)

    acc, den = lax.fori_loop(
        0, TC_NGROUPS, group, (jnp.zeros((1, VDIM), jnp.float32), jnp.float32(0.0))
    )
    oacc_ref[...] = acc
    oden_ref[...] = jnp.full((1, 128), den, jnp.float32)


def _combine_kernel(fz_ref, tacc_ref, tden_ref, sparts_ref, sdens_ref, o_ref):
    acc = tacc_ref[...] + jnp.sum(sparts_ref[...], axis=0, keepdims=True)
    den = tden_ref[0, 0] + jnp.sum(sdens_ref[...])
    o_ref[...] = 0.7 * (acc / den) + 0.3 * fz_ref[...]


@jax.jit
def _soft_read(x_key, f_z_value, key_memory, value_memory):
    m, kdim = key_memory.shape
    v2d = value_memory.reshape(m, VDIM)
    fz2d = f_z_value.reshape(1, VDIM)

    sc_parts, sc_dens = _sc_soft_read(x_key, key_memory, v2d)

    tc_acc, tc_den = pl.pallas_call(
        _tc_soft_read_kernel,
        in_specs=[
            pl.BlockSpec(memory_space=pltpu.MemorySpace.VMEM),
            pl.BlockSpec(memory_space=pltpu.MemorySpace.HBM),
            pl.BlockSpec(memory_space=pltpu.MemorySpace.HBM),
        ],
        out_specs=[
            pl.BlockSpec(memory_space=pltpu.MemorySpace.VMEM),
            pl.BlockSpec(memory_space=pltpu.MemorySpace.VMEM),
        ],
        out_shape=[
            jax.ShapeDtypeStruct((1, VDIM), jnp.float32),
            jax.ShapeDtypeStruct((1, 128), jnp.float32),
        ],
        scratch_shapes=(
            [pltpu.VMEM((TC_CHUNK, KDIM), jnp.float32) for _ in range(TC_NBUF)]
            + [pltpu.VMEM((TC_CHUNK, VDIM), jnp.float32) for _ in range(TC_NBUF)]
            + [pltpu.SemaphoreType.DMA((TC_NBUF, 2))]
        ),
    )(x_key, key_memory, v2d)

    out = pl.pallas_call(
        _combine_kernel,
        out_shape=jax.ShapeDtypeStruct((1, VDIM), jnp.float32),
    )(fz2d, tc_acc, tc_den, sc_parts, sc_dens)
    return out.reshape(f_z_value.shape)


def kernel(x_key, f_z_value, key_memory, value_memory):
    return _soft_read(x_key, f_z_value, key_memory, value_memory)
